# Initial kernel scaffold; baseline (speedup 1.0000x reference)
import jax
import jax.numpy as jnp
from jax.experimental import pallas as pl


def kernel(nodes_representations, hyperedge_arg_node_idxs, unq_hyperedge_type_reprs, hyperedge_type_name_unq_idxs, unq_hyperedge_arg_name_reprs, hyperedge_arg_name_unq_idxs, hyperedge_arg_to_edge_id, num_edges, W1, b1, W2, b2):
    raise NotImplementedError("write your pallas kernel here")


# speedup vs baseline: 2.5343x; 2.5343x over previous
import functools

import jax
import jax.numpy as jnp
from jax import lax
from jax.experimental import pallas as pl
from jax.experimental.pallas import tpu as pltpu
from jax.experimental.pallas import tpu_sc as plsc

H = 128
F = 16
NLANE = 16
NJ = H // NLANE
NEG = float(-3.0e38)

NC = 2
NS = 16
NW = NC * NS

C = 128


def _leaky(x):
    return jnp.maximum(x, 0.01 * x)


def _tables_body(nodes_ref, names_ref, types_ref, w1n_ref, w1d_ref, w1t_ref,
                 w2n_ref, w2d_ref, node1_ref, node2_ref, name1_ref, name2_ref,
                 type1_ref):
    f32 = jnp.float32
    nodes = nodes_ref[...]
    node1_ref[...] = jnp.dot(nodes, w1d_ref[...], preferred_element_type=f32)
    node2_ref[...] = jnp.dot(nodes, w2d_ref[...], preferred_element_type=f32)
    name1_ref[...] = jnp.dot(names_ref[...], w1n_ref[...], preferred_element_type=f32)
    name2_ref[...] = jnp.dot(names_ref[...], w2n_ref[...], preferred_element_type=f32)
    type1_ref[...] = jnp.dot(types_ref[...], w1t_ref[...], preferred_element_type=f32)


def _make_tables(nodes, names, types, W1, W2):
    N = nodes.shape[0]
    T = types.shape[0]
    NAMES = names.shape[0]
    f32 = jnp.float32
    out_shape = [
        jax.ShapeDtypeStruct((N, H), f32),
        jax.ShapeDtypeStruct((N, H), f32),
        jax.ShapeDtypeStruct((NAMES, H), f32),
        jax.ShapeDtypeStruct((NAMES, H), f32),
        jax.ShapeDtypeStruct((T, H), f32),
    ]
    return pl.pallas_call(_tables_body, out_shape=out_shape)(
        nodes, names, types, W1[:F], W1[F:], W1[:H], W2[H:H + F], W2[H + F:])


def _edge_body(pre_ref, t_ref, type1_ref, b1_ref, w2e_ref, b2_ref,
               es_ref, ep_ref):
    f32 = jnp.float32
    be = pre_ref.shape[0]
    T = type1_ref.shape[0]
    tb = t_ref[0, 0, :]
    oh = (tb[:, None] == lax.broadcasted_iota(jnp.int32, (be, T), 1)).astype(f32)
    typ = jnp.dot(oh, type1_ref[...], preferred_element_type=f32)
    pre = jnp.maximum(pre_ref[...], typ) + b1_ref[...]
    es = _leaky(pre)
    es_ref[...] = es
    ep_ref[...] = jnp.dot(es, w2e_ref[...], preferred_element_type=f32) + b2_ref[...]


def _edge_update(premax, t_idx, type1, b1, W2e, b2, BE=16000):
    E = t_idx.shape[0]
    T = type1.shape[0]
    nb = E // BE
    f32 = jnp.float32
    t3 = t_idx.reshape(nb, 1, BE)
    grid = (nb,)
    out_shape = [jax.ShapeDtypeStruct((E, H), f32),
                 jax.ShapeDtypeStruct((E, H), f32)]
    return pl.pallas_call(
        _edge_body,
        grid=grid,
        in_specs=[
            pl.BlockSpec((BE, H), lambda i: (i, 0)),
            pl.BlockSpec((1, 1, BE), lambda i: (i, 0, 0)),
            pl.BlockSpec((T, H), lambda i: (0, 0)),
            pl.BlockSpec((1, H), lambda i: (0, 0)),
            pl.BlockSpec((H, H), lambda i: (0, 0)),
            pl.BlockSpec((1, H), lambda i: (0, 0)),
        ],
        out_specs=[
            pl.BlockSpec((BE, H), lambda i: (i, 0)),
            pl.BlockSpec((BE, H), lambda i: (i, 0)),
        ],
        out_shape=out_shape,
    )(premax, t3, type1, b1.reshape(1, H), W2e, b2.reshape(1, H))


def _segmax_call(eid_p, nm_p, nd_p, qtab, name1, node1, Epad, CE, CPT):
    f32 = jnp.float32
    i32 = jnp.int32
    mesh = plsc.VectorSubcoreMesh(core_axis_name="c", subcore_axis_name="s")

    @functools.partial(
        pl.kernel,
        out_type=jax.ShapeDtypeStruct((Epad, H), f32),
        mesh=mesh,
        scratch_types=[
            pltpu.VMEM((48,), i32),
            pltpu.VMEM((C,), i32),
            pltpu.VMEM((C,), i32),
            pltpu.VMEM((C,), i32),
            pltpu.VMEM((C, H), f32),
            pltpu.VMEM((C, H), f32),
            pltpu.VMEM((CE, H), f32),
            pltpu.SemaphoreType.DMA,
            pltpu.SemaphoreType.DMA,
        ],
    )
    def seg(eid_hbm, nm_hbm, nd_hbm, qtab_hbm, name1_hbm, node1_hbm, pre_hbm,
            qbuf, eidb, nmb, ndb, namerows, noderows, out_local, sem1, sem2):
        wid = lax.axis_index("s") * NC + lax.axis_index("c")
        pltpu.sync_copy(qtab_hbm.at[wid], qbuf)
        neg16 = jnp.full((NLANE,), NEG, f32)

        def chunk_body(c, _):
            ce0 = (wid * CPT + c) * CE
            qa = qbuf[pl.ds(c, 16)][0]
            qe = qbuf[pl.ds(16 + c, 16)][0]

            def init_body(r, _):
                for j in range(NJ):
                    out_local[r, NLANE * j:NLANE * (j + 1)] = neg16
                return 0
            lax.fori_loop(0, CE, init_body, 0)

            nsub = (qe - qa + (C - 1)) >> 7

            def sub_body(s, carry):
                p = pl.multiple_of(qa + s * C, 8)
                pltpu.sync_copy(eid_hbm.at[pl.ds(p, C)], eidb)
                pltpu.sync_copy(nm_hbm.at[pl.ds(p, C)], nmb)
                pltpu.sync_copy(nd_hbm.at[pl.ds(p, C)], ndb)
                d1 = pltpu.async_copy(name1_hbm.at[nmb], namerows, sem1)
                d2 = pltpu.async_copy(node1_hbm.at[ndb], noderows, sem2)
                d1.wait()
                d2.wait()
                nv = jnp.minimum(C, qe - p)
                ngrp = (nv + 15) >> 4

                def grp_body(g, gcarry):
                    prev_le = gcarry[0]
                    acc = gcarry[1]
                    r0 = g * 16
                    egrp = eidb[pl.ds(r0, 16)]
                    for ii in range(16):
                        le = egrp[ii] - ce0
                        valid = (r0 + ii) < nv
                        d = jnp.where(le == prev_le, jnp.float32(0.0),
                                      jnp.float32(NEG))
                        dv = jnp.full((NLANE,), d, f32)
                        newacc = []
                        for j in range(NJ):
                            sl = slice(NLANE * j, NLANE * (j + 1))
                            z = namerows[r0 + ii, sl] + noderows[r0 + ii, sl]
                            newacc.append(jnp.maximum(acc[j] + dv, z))
                        acc = tuple(newacc)

                        @pl.when(jnp.logical_and(valid, le >= 0))
                        def _(acc=acc, le=le):
                            for j in range(NJ):
                                out_local[le, NLANE * j:NLANE * (j + 1)] = acc[j]
                        prev_le = le
                    return (prev_le, acc)

                return lax.fori_loop(0, ngrp, grp_body, carry)

            acc0 = tuple(neg16 for _ in range(NJ))
            lax.fori_loop(0, nsub, sub_body, (jnp.int32(-2147483600), acc0))
            pltpu.sync_copy(out_local, pre_hbm.at[pl.ds(ce0, CE)])
            return 0

        lax.fori_loop(0, CPT, chunk_body, 0)

    return seg(eid_p, nm_p, nd_p, qtab, name1, node1)


def _final_call(eid_p, nm_p, nd_p, ep2b, name2, node2, A):
    f32 = jnp.float32
    i32 = jnp.int32
    APT = A // NW
    NSUB = (APT + C - 1) // C
    LASTO = APT - C
    mesh = plsc.VectorSubcoreMesh(core_axis_name="c", subcore_axis_name="s")

    @functools.partial(
        pl.kernel,
        out_type=jax.ShapeDtypeStruct((A, H), f32),
        mesh=mesh,
        scratch_types=[
            pltpu.VMEM((C,), i32),
            pltpu.VMEM((C,), i32),
            pltpu.VMEM((C,), i32),
            pltpu.VMEM((C, H), f32),
            pltpu.VMEM((C, H), f32),
            pltpu.VMEM((C, H), f32),
            pltpu.VMEM((C, H), f32),
            pltpu.SemaphoreType.DMA,
            pltpu.SemaphoreType.DMA,
            pltpu.SemaphoreType.DMA,
        ],
    )
    def fin(eid_hbm, nm_hbm, nd_hbm, ep_hbm, name2_hbm, node2_hbm, out_hbm,
            eidb, nmb, ndb, eprows, namerows, noderows, outloc,
            sem1, sem2, sem3):
        wid = lax.axis_index("s") * NC + lax.axis_index("c")
        base = wid * APT

        def chunk_body(k, _):
            o = pl.multiple_of(base + jnp.minimum(k * C, LASTO), 8)
            pltpu.sync_copy(eid_hbm.at[pl.ds(o, C)], eidb)
            pltpu.sync_copy(nm_hbm.at[pl.ds(o, C)], nmb)
            pltpu.sync_copy(nd_hbm.at[pl.ds(o, C)], ndb)
            d1 = pltpu.async_copy(ep_hbm.at[eidb], eprows, sem1)
            d2 = pltpu.async_copy(name2_hbm.at[nmb], namerows, sem2)
            d3 = pltpu.async_copy(node2_hbm.at[ndb], noderows, sem3)
            d1.wait()
            d2.wait()
            d3.wait()

            def row_body(i, _):
                for j in range(NJ):
                    sl = slice(NLANE * j, NLANE * (j + 1))
                    x = eprows[i, sl] + namerows[i, sl] + noderows[i, sl]
                    outloc[i, sl] = jnp.maximum(x, 0.01 * x)
                return 0
            lax.fori_loop(0, C, row_body, 0)
            pltpu.sync_copy(outloc, out_hbm.at[pl.ds(o, C)])
            return 0

        lax.fori_loop(0, NSUB, chunk_body, 0)

    return fin(eid_p, nm_p, nd_p, ep2b, name2, node2)


def kernel(nodes_representations, hyperedge_arg_node_idxs,
           unq_hyperedge_type_reprs, hyperedge_type_name_unq_idxs,
           unq_hyperedge_arg_name_reprs, hyperedge_arg_name_unq_idxs,
           hyperedge_arg_to_edge_id, num_edges, W1, b1, W2, b2):
    A = hyperedge_arg_to_edge_id.shape[0]
    E = hyperedge_type_name_unq_idxs.shape[0]
    i32 = jnp.int32

    CE = 256
    CPT = -(-E // (NW * CE))
    NCHUNK = NW * CPT
    Epad = NCHUNK * CE

    node1, node2, name1, name2, type1 = _make_tables(
        nodes_representations, unq_hyperedge_arg_name_reprs,
        unq_hyperedge_type_reprs, W1, W2)

    eid = hyperedge_arg_to_edge_id
    bounds = jnp.arange(0, Epad + 1, CE, dtype=i32)
    q = jnp.searchsorted(eid, bounds[:NCHUNK + 1], side="left").astype(i32)
    qa = (q[:-1] // 8) * 8
    qtab = jnp.concatenate([
        jnp.pad(qa.reshape(NW, CPT), ((0, 0), (0, 16 - CPT))),
        jnp.pad(q[1:].reshape(NW, CPT), ((0, 0), (0, 32 - CPT))),
    ], axis=1)

    pad = jnp.zeros((C,), i32)
    eid_p = jnp.concatenate([eid, pad])
    nm_p = jnp.concatenate([hyperedge_arg_name_unq_idxs, pad])
    nd_p = jnp.concatenate([hyperedge_arg_node_idxs, pad])

    premax = _segmax_call(eid_p, nm_p, nd_p, qtab, name1, node1, Epad, CE, CPT)

    edge_states, ep2b = _edge_update(
        premax, hyperedge_type_name_unq_idxs, type1, b1, W2[:H], b2)

    msgs = _final_call(eid_p, nm_p, nd_p, ep2b, name2, node2, A)

    return (msgs, edge_states)


# speedup vs baseline: 2.5646x; 1.0120x over previous
import functools

import jax
import jax.numpy as jnp
from jax import lax
from jax.experimental import pallas as pl
from jax.experimental.pallas import tpu as pltpu
from jax.experimental.pallas import tpu_sc as plsc

H = 128
F = 16
NLANE = 16
NJ = H // NLANE
NEG = float(-3.0e38)

NC = 2
NS = 16
NW = NC * NS

C = 128


def _leaky(x):
    return jnp.maximum(x, 0.01 * x)


def _tables_body(nodes_ref, names_ref, types_ref, w1n_ref, w1d_ref, w1t_ref,
                 w2n_ref, w2d_ref, node1_ref, node2_ref, name1_ref, name2_ref,
                 type1_ref):
    f32 = jnp.float32
    nodes = nodes_ref[...]
    node1_ref[...] = jnp.dot(nodes, w1d_ref[...], preferred_element_type=f32)
    node2_ref[...] = jnp.dot(nodes, w2d_ref[...], preferred_element_type=f32)
    name1_ref[...] = jnp.dot(names_ref[...], w1n_ref[...], preferred_element_type=f32)
    name2_ref[...] = jnp.dot(names_ref[...], w2n_ref[...], preferred_element_type=f32)
    type1_ref[...] = jnp.dot(types_ref[...], w1t_ref[...], preferred_element_type=f32)


def _make_tables(nodes, names, types, W1, W2):
    N = nodes.shape[0]
    T = types.shape[0]
    NAMES = names.shape[0]
    f32 = jnp.float32
    out_shape = [
        jax.ShapeDtypeStruct((N, H), f32),
        jax.ShapeDtypeStruct((N, H), f32),
        jax.ShapeDtypeStruct((NAMES, H), f32),
        jax.ShapeDtypeStruct((NAMES, H), f32),
        jax.ShapeDtypeStruct((T, H), f32),
    ]
    return pl.pallas_call(_tables_body, out_shape=out_shape)(
        nodes, names, types, W1[:F], W1[F:], W1[:H], W2[H:H + F], W2[H + F:])


def _edge_body(pre_ref, t_ref, type1_ref, b1_ref, w2e_ref, b2_ref,
               es_ref, ep_ref):
    f32 = jnp.float32
    be = pre_ref.shape[0]
    T = type1_ref.shape[0]
    tb = t_ref[0, 0, :]
    oh = (tb[:, None] == lax.broadcasted_iota(jnp.int32, (be, T), 1)).astype(f32)
    typ = jnp.dot(oh, type1_ref[...], preferred_element_type=f32)
    pre = jnp.maximum(pre_ref[...], typ) + b1_ref[...]
    es = _leaky(pre)
    es_ref[...] = es
    ep_ref[...] = jnp.dot(es, w2e_ref[...], preferred_element_type=f32) + b2_ref[...]


def _edge_update(premax, t_idx, type1, b1, W2e, b2, BE=16000):
    E = t_idx.shape[0]
    T = type1.shape[0]
    nb = E // BE
    f32 = jnp.float32
    t3 = t_idx.reshape(nb, 1, BE)
    grid = (nb,)
    out_shape = [jax.ShapeDtypeStruct((E, H), f32),
                 jax.ShapeDtypeStruct((E, H), f32)]
    return pl.pallas_call(
        _edge_body,
        grid=grid,
        in_specs=[
            pl.BlockSpec((BE, H), lambda i: (i, 0)),
            pl.BlockSpec((1, 1, BE), lambda i: (i, 0, 0)),
            pl.BlockSpec((T, H), lambda i: (0, 0)),
            pl.BlockSpec((1, H), lambda i: (0, 0)),
            pl.BlockSpec((H, H), lambda i: (0, 0)),
            pl.BlockSpec((1, H), lambda i: (0, 0)),
        ],
        out_specs=[
            pl.BlockSpec((BE, H), lambda i: (i, 0)),
            pl.BlockSpec((BE, H), lambda i: (i, 0)),
        ],
        out_shape=out_shape,
    )(premax, t3, type1, b1.reshape(1, H), W2e, b2.reshape(1, H))


def _segmax_call(eid_p, nm_p, nd_p, qtab, name1, node1, Epad, CE, CPT):
    f32 = jnp.float32
    i32 = jnp.int32
    mesh = plsc.VectorSubcoreMesh(core_axis_name="c", subcore_axis_name="s")

    @functools.partial(
        pl.kernel,
        out_type=jax.ShapeDtypeStruct((Epad, H), f32),
        mesh=mesh,
        scratch_types=[
            pltpu.VMEM((48,), i32),
            pltpu.VMEM((C,), i32),
            pltpu.VMEM((C,), i32),
            pltpu.VMEM((C,), i32),
            pltpu.VMEM((C, H), f32),
            pltpu.VMEM((C, H), f32),
            pltpu.VMEM((CE, H), f32),
            pltpu.SemaphoreType.DMA,
            pltpu.SemaphoreType.DMA,
        ],
    )
    def seg(eid_hbm, nm_hbm, nd_hbm, qtab_hbm, name1_hbm, node1_hbm, pre_hbm,
            qbuf, eidb, nmb, ndb, namerows, noderows, out_local, sem1, sem2):
        wid = lax.axis_index("s") * NC + lax.axis_index("c")
        pltpu.sync_copy(qtab_hbm.at[wid], qbuf)
        neg16 = jnp.full((NLANE,), NEG, f32)

        def chunk_body(c, _):
            ce0 = (wid * CPT + c) * CE
            qa = qbuf[pl.ds(c, 16)][0]
            qe = qbuf[pl.ds(16 + c, 16)][0]

            def init_body(r, _):
                for j in range(NJ):
                    out_local[r, NLANE * j:NLANE * (j + 1)] = neg16
                return 0
            lax.fori_loop(0, CE, init_body, 0)

            nsub = (qe - qa + (C - 1)) >> 7

            def sub_body(s, carry):
                p = pl.multiple_of(qa + s * C, 8)
                pltpu.sync_copy(eid_hbm.at[pl.ds(p, C)], eidb)
                pltpu.sync_copy(nm_hbm.at[pl.ds(p, C)], nmb)
                pltpu.sync_copy(nd_hbm.at[pl.ds(p, C)], ndb)
                d1 = pltpu.async_copy(name1_hbm.at[nmb], namerows, sem1)
                d2 = pltpu.async_copy(node1_hbm.at[ndb], noderows, sem2)
                d1.wait()
                d2.wait()
                nv = jnp.minimum(C, qe - p)
                ngrp = (nv + 15) >> 4

                def grp_body(g, gcarry):
                    prev_le = gcarry[0]
                    acc = gcarry[1]
                    r0 = g * 16
                    egrp = eidb[pl.ds(r0, 16)]
                    for ii in range(16):
                        le = egrp[ii] - ce0
                        valid = (r0 + ii) < nv
                        d = jnp.where(le == prev_le, jnp.float32(0.0),
                                      jnp.float32(NEG))
                        dv = jnp.full((NLANE,), d, f32)
                        newacc = []
                        for j in range(NJ):
                            sl = slice(NLANE * j, NLANE * (j + 1))
                            z = namerows[r0 + ii, sl] + noderows[r0 + ii, sl]
                            newacc.append(jnp.maximum(acc[j] + dv, z))
                        acc = tuple(newacc)

                        @pl.when(jnp.logical_and(valid, le >= 0))
                        def _(acc=acc, le=le):
                            for j in range(NJ):
                                out_local[le, NLANE * j:NLANE * (j + 1)] = acc[j]
                        prev_le = le
                    return (prev_le, acc)

                return lax.fori_loop(0, ngrp, grp_body, carry)

            acc0 = tuple(neg16 for _ in range(NJ))
            lax.fori_loop(0, nsub, sub_body, (jnp.int32(-2147483600), acc0))
            pltpu.sync_copy(out_local, pre_hbm.at[pl.ds(ce0, CE)])
            return 0

        lax.fori_loop(0, CPT, chunk_body, 0)

    return seg(eid_p, nm_p, nd_p, qtab, name1, node1)


def _final_call(eid_p, nm_p, nd_p, ep2b, name2, node2, A):
    f32 = jnp.float32
    i32 = jnp.int32
    APT = A // NW
    NSUB = (APT + C - 1) // C
    NITER = 2 * ((NSUB + 1) // 2)
    LASTO = APT - C
    mesh = plsc.VectorSubcoreMesh(core_axis_name="c", subcore_axis_name="s")

    @functools.partial(
        pl.kernel,
        out_type=jax.ShapeDtypeStruct((A, H), f32),
        mesh=mesh,
        scratch_types=[
            [pltpu.VMEM((C,), i32)] * 2,
            [pltpu.VMEM((C,), i32)] * 2,
            [pltpu.VMEM((C,), i32)] * 2,
            [pltpu.VMEM((C, H), f32)] * 2,
            [pltpu.VMEM((C, H), f32)] * 2,
            [pltpu.VMEM((C, H), f32)] * 2,
            [pltpu.SemaphoreType.DMA] * 2,
            [pltpu.SemaphoreType.DMA] * 2,
            [pltpu.SemaphoreType.DMA] * 2,
        ],
    )
    def fin(eid_hbm, nm_hbm, nd_hbm, ep_hbm, name2_hbm, node2_hbm, out_hbm,
            eidb, nmb, ndb, eprows, namerows, noderows, semi, semg, semst):
        wid = lax.axis_index("s") * NC + lax.axis_index("c")
        base = wid * APT

        def off(k):
            return pl.multiple_of(base + jnp.minimum(k * C, LASTO), 8)

        def idx_copies(k, s):
            o = off(k)
            return (pltpu.make_async_copy(eid_hbm.at[pl.ds(o, C)], eidb[s], semi[s]),
                    pltpu.make_async_copy(nm_hbm.at[pl.ds(o, C)], nmb[s], semi[s]),
                    pltpu.make_async_copy(nd_hbm.at[pl.ds(o, C)], ndb[s], semi[s]))

        def gathers(s):
            return (pltpu.make_async_copy(ep_hbm.at[eidb[s]], eprows[s], semg[s]),
                    pltpu.make_async_copy(name2_hbm.at[nmb[s]], namerows[s], semg[s]),
                    pltpu.make_async_copy(node2_hbm.at[ndb[s]], noderows[s], semg[s]))

        def store(k, s):
            return pltpu.make_async_copy(eprows[s], out_hbm.at[pl.ds(off(k), C)],
                                         semst[s])

        def issue(ds):
            for d in ds:
                d.start()

        def drain(ds):
            for d in ds:
                d.wait()

        drain_issue0 = idx_copies(0, 0)
        issue(drain_issue0)
        drain(drain_issue0)
        issue(gathers(0))
        issue(idx_copies(1, 1))

        def pair_body(p, _):
            for s in range(2):
                k = 2 * p + s
                drain(gathers(s))
                issue(idx_copies(k + 2, s))

                def row_body(i, _):
                    for j in range(NJ):
                        sl = slice(NLANE * j, NLANE * (j + 1))
                        x = eprows[s][i, sl] + namerows[s][i, sl] + noderows[s][i, sl]
                        eprows[s][i, sl] = jnp.maximum(x, 0.01 * x)
                    return 0
                lax.fori_loop(0, C, row_body, 0)
                issue((store(k, s),))
                drain(idx_copies(k + 1, 1 - s))

                @pl.when(k >= 1)
                def _():
                    drain((store(k - 1, 1 - s),))
                issue(gathers(1 - s))
            return 0

        lax.fori_loop(0, NITER // 2, pair_body, 0)
        drain(gathers(0))
        drain(idx_copies(NITER + 1, 1))
        drain((store(NITER - 1, 1),))

    return fin(eid_p, nm_p, nd_p, ep2b, name2, node2)


def kernel(nodes_representations, hyperedge_arg_node_idxs,
           unq_hyperedge_type_reprs, hyperedge_type_name_unq_idxs,
           unq_hyperedge_arg_name_reprs, hyperedge_arg_name_unq_idxs,
           hyperedge_arg_to_edge_id, num_edges, W1, b1, W2, b2):
    A = hyperedge_arg_to_edge_id.shape[0]
    E = hyperedge_type_name_unq_idxs.shape[0]
    i32 = jnp.int32

    CE = 256
    CPT = -(-E // (NW * CE))
    NCHUNK = NW * CPT
    Epad = NCHUNK * CE

    node1, node2, name1, name2, type1 = _make_tables(
        nodes_representations, unq_hyperedge_arg_name_reprs,
        unq_hyperedge_type_reprs, W1, W2)

    eid = hyperedge_arg_to_edge_id
    bounds = jnp.arange(0, Epad + 1, CE, dtype=i32)
    q = jnp.searchsorted(eid, bounds[:NCHUNK + 1], side="left").astype(i32)
    qa = (q[:-1] // 8) * 8
    qtab = jnp.concatenate([
        jnp.pad(qa.reshape(NW, CPT), ((0, 0), (0, 16 - CPT))),
        jnp.pad(q[1:].reshape(NW, CPT), ((0, 0), (0, 32 - CPT))),
    ], axis=1)

    pad = jnp.zeros((C,), i32)
    eid_p = jnp.concatenate([eid, pad])
    nm_p = jnp.concatenate([hyperedge_arg_name_unq_idxs, pad])
    nd_p = jnp.concatenate([hyperedge_arg_node_idxs, pad])

    premax = _segmax_call(eid_p, nm_p, nd_p, qtab, name1, node1, Epad, CE, CPT)

    edge_states, ep2b = _edge_update(
        premax, hyperedge_type_name_unq_idxs, type1, b1, W2[:H], b2)

    msgs = _final_call(eid_p, nm_p, nd_p, ep2b, name2, node2, A)

    return (msgs, edge_states)


# speedup vs baseline: 2.8615x; 1.1158x over previous
import functools

import jax
import jax.numpy as jnp
from jax import lax
from jax.experimental import pallas as pl
from jax.experimental.pallas import tpu as pltpu
from jax.experimental.pallas import tpu_sc as plsc

H = 128
F = 16
NLANE = 16
NJ = H // NLANE
NEG = float(-3.0e38)

NC = 2
NS = 16
NW = NC * NS

C = 128


def _leaky(x):
    return jnp.maximum(x, 0.01 * x)


def _tables_body(nodes_ref, names_ref, types_ref, w1n_ref, w1d_ref, w1t_ref,
                 w2n_ref, w2d_ref, node1_ref, node2_ref, name1_ref, name2_ref,
                 type1_ref):
    f32 = jnp.float32
    nodes = nodes_ref[...]
    node1_ref[...] = jnp.dot(nodes, w1d_ref[...], preferred_element_type=f32)
    node2_ref[...] = jnp.dot(nodes, w2d_ref[...], preferred_element_type=f32)
    name1_ref[...] = jnp.dot(names_ref[...], w1n_ref[...], preferred_element_type=f32)
    name2_ref[...] = jnp.dot(names_ref[...], w2n_ref[...], preferred_element_type=f32)
    type1_ref[...] = jnp.dot(types_ref[...], w1t_ref[...], preferred_element_type=f32)


def _make_tables(nodes, names, types, W1, W2):
    N = nodes.shape[0]
    T = types.shape[0]
    NAMES = names.shape[0]
    f32 = jnp.float32
    out_shape = [
        jax.ShapeDtypeStruct((N, H), f32),
        jax.ShapeDtypeStruct((N, H), f32),
        jax.ShapeDtypeStruct((NAMES, H), f32),
        jax.ShapeDtypeStruct((NAMES, H), f32),
        jax.ShapeDtypeStruct((T, H), f32),
    ]
    return pl.pallas_call(_tables_body, out_shape=out_shape)(
        nodes, names, types, W1[:F], W1[F:], W1[:H], W2[H:H + F], W2[H + F:])


def _edge_body(pre_ref, t_ref, type1_ref, b1_ref, w2e_ref, b2_ref,
               es_ref, ep_ref):
    f32 = jnp.float32
    be = pre_ref.shape[0]
    T = type1_ref.shape[0]
    tb = t_ref[0, 0, :]
    oh = (tb[:, None] == lax.broadcasted_iota(jnp.int32, (be, T), 1)).astype(f32)
    typ = jnp.dot(oh, type1_ref[...], preferred_element_type=f32)
    pre = jnp.maximum(pre_ref[...], typ) + b1_ref[...]
    es = _leaky(pre)
    es_ref[...] = es
    ep_ref[...] = jnp.dot(es, w2e_ref[...], preferred_element_type=f32) + b2_ref[...]


def _edge_update(premax, t_idx, type1, b1, W2e, b2, BE=16000):
    E = t_idx.shape[0]
    T = type1.shape[0]
    nb = E // BE
    f32 = jnp.float32
    t3 = t_idx.reshape(nb, 1, BE)
    grid = (nb,)
    out_shape = [jax.ShapeDtypeStruct((E, H), f32),
                 jax.ShapeDtypeStruct((E, H), f32)]
    return pl.pallas_call(
        _edge_body,
        grid=grid,
        in_specs=[
            pl.BlockSpec((BE, H), lambda i: (i, 0)),
            pl.BlockSpec((1, 1, BE), lambda i: (i, 0, 0)),
            pl.BlockSpec((T, H), lambda i: (0, 0)),
            pl.BlockSpec((1, H), lambda i: (0, 0)),
            pl.BlockSpec((H, H), lambda i: (0, 0)),
            pl.BlockSpec((1, H), lambda i: (0, 0)),
        ],
        out_specs=[
            pl.BlockSpec((BE, H), lambda i: (i, 0)),
            pl.BlockSpec((BE, H), lambda i: (i, 0)),
        ],
        out_shape=out_shape,
    )(premax, t3, type1, b1.reshape(1, H), W2e, b2.reshape(1, H))


def _segmax_call(eid_p, nm_p, nd_p, qtab, name1, node1, Epad, CE, CPT):
    f32 = jnp.float32
    i32 = jnp.int32
    mesh = plsc.VectorSubcoreMesh(core_axis_name="c", subcore_axis_name="s")

    @functools.partial(
        pl.kernel,
        out_type=jax.ShapeDtypeStruct((Epad, H), f32),
        mesh=mesh,
        scratch_types=[
            pltpu.VMEM((48,), i32),
            pltpu.VMEM((C,), i32),
            pltpu.VMEM((C,), i32),
            pltpu.VMEM((C,), i32),
            pltpu.VMEM((32, H), f32),
            pltpu.VMEM((C, H), f32),
            pltpu.VMEM((CE, H), f32),
            pltpu.SemaphoreType.DMA,
            pltpu.SemaphoreType.DMA,
        ],
    )
    def seg(eid_hbm, nm_hbm, nd_hbm, qtab_hbm, name1_hbm, node1_hbm, pre_hbm,
            qbuf, eidb, nmb, ndb, name1loc, noderows, out_local, sem1, sem2):
        wid = lax.axis_index("s") * NC + lax.axis_index("c")
        pltpu.sync_copy(qtab_hbm.at[wid], qbuf)
        pltpu.sync_copy(name1_hbm, name1loc)
        neg16 = jnp.full((NLANE,), NEG, f32)

        def chunk_body(c, _):
            ce0 = (wid * CPT + c) * CE
            qa = qbuf[pl.ds(c, 16)][0]
            qe = qbuf[pl.ds(16 + c, 16)][0]

            def init_body(r, _):
                for j in range(NJ):
                    out_local[r, NLANE * j:NLANE * (j + 1)] = neg16
                return 0
            lax.fori_loop(0, CE, init_body, 0)

            nsub = (qe - qa + (C - 1)) >> 7

            def sub_body(s, carry):
                p = pl.multiple_of(qa + s * C, 8)
                pltpu.sync_copy(eid_hbm.at[pl.ds(p, C)], eidb)
                pltpu.sync_copy(nm_hbm.at[pl.ds(p, C)], nmb)
                pltpu.sync_copy(nd_hbm.at[pl.ds(p, C)], ndb)
                d2 = pltpu.async_copy(node1_hbm.at[ndb], noderows, sem2)
                d2.wait()
                nv = jnp.minimum(C, qe - p)
                ngrp = (nv + 15) >> 4

                def grp_body(g, gcarry):
                    prev_le = gcarry[0]
                    acc = gcarry[1]
                    r0 = g * 16
                    egrp = eidb[pl.ds(r0, 16)]
                    nmg = nmb[pl.ds(r0, 16)]
                    for ii in range(16):
                        le = egrp[ii] - ce0
                        nmi = nmg[ii]
                        valid = (r0 + ii) < nv
                        d = jnp.where(le == prev_le, jnp.float32(0.0),
                                      jnp.float32(NEG))
                        dv = jnp.full((NLANE,), d, f32)
                        newacc = []
                        for j in range(NJ):
                            sl = slice(NLANE * j, NLANE * (j + 1))
                            z = name1loc[nmi, sl] + noderows[r0 + ii, sl]
                            newacc.append(jnp.maximum(acc[j] + dv, z))
                        acc = tuple(newacc)

                        @pl.when(jnp.logical_and(valid, le >= 0))
                        def _(acc=acc, le=le):
                            for j in range(NJ):
                                out_local[le, NLANE * j:NLANE * (j + 1)] = acc[j]
                        prev_le = le
                    return (prev_le, acc)

                return lax.fori_loop(0, ngrp, grp_body, carry)

            acc0 = tuple(neg16 for _ in range(NJ))
            lax.fori_loop(0, nsub, sub_body, (jnp.int32(-2147483600), acc0))
            pltpu.sync_copy(out_local, pre_hbm.at[pl.ds(ce0, CE)])
            return 0

        lax.fori_loop(0, CPT, chunk_body, 0)

    return seg(eid_p, nm_p, nd_p, qtab, name1, node1)


def _final_call(eid_p, nm_p, nd_p, ep2b, name2, node2, A):
    f32 = jnp.float32
    i32 = jnp.int32
    APT = A // NW
    NSUB = (APT + C - 1) // C
    NITER = 2 * ((NSUB + 1) // 2)
    LASTO = APT - C
    mesh = plsc.VectorSubcoreMesh(core_axis_name="c", subcore_axis_name="s")

    @functools.partial(
        pl.kernel,
        out_type=jax.ShapeDtypeStruct((A, H), f32),
        mesh=mesh,
        scratch_types=[
            [pltpu.VMEM((C,), i32)] * 2,
            [pltpu.VMEM((C,), i32)] * 2,
            [pltpu.VMEM((C,), i32)] * 2,
            [pltpu.VMEM((C, H), f32)] * 2,
            pltpu.VMEM((32, H), f32),
            [pltpu.VMEM((C, H), f32)] * 2,
            [pltpu.SemaphoreType.DMA] * 2,
            [pltpu.SemaphoreType.DMA] * 2,
            [pltpu.SemaphoreType.DMA] * 2,
        ],
    )
    def fin(eid_hbm, nm_hbm, nd_hbm, ep_hbm, name2_hbm, node2_hbm, out_hbm,
            eidb, nmb, ndb, eprows, name2loc, noderows, semi, semg, semst):
        wid = lax.axis_index("s") * NC + lax.axis_index("c")
        base = wid * APT

        def off(k):
            return pl.multiple_of(base + jnp.minimum(k * C, LASTO), 8)

        def idx_copies(k, s):
            o = off(k)
            return (pltpu.make_async_copy(eid_hbm.at[pl.ds(o, C)], eidb[s], semi[s]),
                    pltpu.make_async_copy(nm_hbm.at[pl.ds(o, C)], nmb[s], semi[s]),
                    pltpu.make_async_copy(nd_hbm.at[pl.ds(o, C)], ndb[s], semi[s]))

        def gathers(s):
            return (pltpu.make_async_copy(ep_hbm.at[eidb[s]], eprows[s], semg[s]),
                    pltpu.make_async_copy(node2_hbm.at[ndb[s]], noderows[s], semg[s]))

        def store(k, s):
            return pltpu.make_async_copy(eprows[s], out_hbm.at[pl.ds(off(k), C)],
                                         semst[s])

        def issue(ds):
            for d in ds:
                d.start()

        def drain(ds):
            for d in ds:
                d.wait()

        pltpu.sync_copy(name2_hbm, name2loc)
        drain_issue0 = idx_copies(0, 0)
        issue(drain_issue0)
        drain(drain_issue0)
        issue(gathers(0))
        issue(idx_copies(1, 1))

        def pair_body(p, _):
            for s in range(2):
                k = 2 * p + s
                drain(gathers(s))

                def grp_body(g, _):
                    r0 = g * 16
                    nmg = nmb[s][pl.ds(r0, 16)]
                    for ii in range(16):
                        i = r0 + ii
                        nmi = nmg[ii]
                        for j in range(NJ):
                            sl = slice(NLANE * j, NLANE * (j + 1))
                            x = (eprows[s][i, sl] + name2loc[nmi, sl]
                                 + noderows[s][i, sl])
                            eprows[s][i, sl] = jnp.maximum(x, 0.01 * x)
                    return 0
                lax.fori_loop(0, C // 16, grp_body, 0)
                issue((store(k, s),))
                issue(idx_copies(k + 2, s))
                drain(idx_copies(k + 1, 1 - s))

                @pl.when(k >= 1)
                def _():
                    drain((store(k - 1, 1 - s),))
                issue(gathers(1 - s))
            return 0

        lax.fori_loop(0, NITER // 2, pair_body, 0)
        drain(gathers(0))
        drain(idx_copies(NITER + 1, 1))
        drain((store(NITER - 1, 1),))

    return fin(eid_p, nm_p, nd_p, ep2b, name2, node2)


def kernel(nodes_representations, hyperedge_arg_node_idxs,
           unq_hyperedge_type_reprs, hyperedge_type_name_unq_idxs,
           unq_hyperedge_arg_name_reprs, hyperedge_arg_name_unq_idxs,
           hyperedge_arg_to_edge_id, num_edges, W1, b1, W2, b2):
    A = hyperedge_arg_to_edge_id.shape[0]
    E = hyperedge_type_name_unq_idxs.shape[0]
    i32 = jnp.int32

    CE = 256
    CPT = -(-E // (NW * CE))
    NCHUNK = NW * CPT
    Epad = NCHUNK * CE

    node1, node2, name1, name2, type1 = _make_tables(
        nodes_representations, unq_hyperedge_arg_name_reprs,
        unq_hyperedge_type_reprs, W1, W2)

    eid = hyperedge_arg_to_edge_id
    bounds = jnp.arange(0, Epad + 1, CE, dtype=i32)
    q = jnp.searchsorted(eid, bounds[:NCHUNK + 1], side="left").astype(i32)
    qa = (q[:-1] // 8) * 8
    qtab = jnp.concatenate([
        jnp.pad(qa.reshape(NW, CPT), ((0, 0), (0, 16 - CPT))),
        jnp.pad(q[1:].reshape(NW, CPT), ((0, 0), (0, 32 - CPT))),
    ], axis=1)

    pad = jnp.zeros((C,), i32)
    eid_p = jnp.concatenate([eid, pad])
    nm_p = jnp.concatenate([hyperedge_arg_name_unq_idxs, pad])
    nd_p = jnp.concatenate([hyperedge_arg_node_idxs, pad])

    premax = _segmax_call(eid_p, nm_p, nd_p, qtab, name1, node1, Epad, CE, CPT)

    edge_states, ep2b = _edge_update(
        premax, hyperedge_type_name_unq_idxs, type1, b1, W2[:H], b2)

    msgs = _final_call(eid_p, nm_p, nd_p, ep2b, name2, node2, A)

    return (msgs, edge_states)


# speedup vs baseline: 3.0256x; 1.0573x over previous
import functools

import jax
import jax.numpy as jnp
from jax import lax
from jax.experimental import pallas as pl
from jax.experimental.pallas import tpu as pltpu
from jax.experimental.pallas import tpu_sc as plsc

H = 128
F = 16
NLANE = 16
NJ = H // NLANE
NEG = float(-3.0e38)

NC = 2
NS = 16
NW = NC * NS

C = 128


def _leaky(x):
    return jnp.maximum(x, 0.01 * x)


def _tables_body(nodes_ref, names_ref, types_ref, w1n_ref, w1d_ref, w1t_ref,
                 w2n_ref, w2d_ref, node1_ref, node2_ref, name1_ref, name2_ref,
                 type1_ref):
    f32 = jnp.float32
    nodes = nodes_ref[...]
    node1_ref[...] = jnp.dot(nodes, w1d_ref[...], preferred_element_type=f32)
    node2_ref[...] = jnp.dot(nodes, w2d_ref[...], preferred_element_type=f32)
    name1_ref[...] = jnp.dot(names_ref[...], w1n_ref[...], preferred_element_type=f32)
    name2_ref[...] = jnp.dot(names_ref[...], w2n_ref[...], preferred_element_type=f32)
    type1_ref[...] = jnp.dot(types_ref[...], w1t_ref[...], preferred_element_type=f32)


def _make_tables(nodes, names, types, W1, W2):
    N = nodes.shape[0]
    T = types.shape[0]
    NAMES = names.shape[0]
    f32 = jnp.float32
    out_shape = [
        jax.ShapeDtypeStruct((N, H), f32),
        jax.ShapeDtypeStruct((N, H), f32),
        jax.ShapeDtypeStruct((NAMES, H), f32),
        jax.ShapeDtypeStruct((NAMES, H), f32),
        jax.ShapeDtypeStruct((T, H), f32),
    ]
    return pl.pallas_call(_tables_body, out_shape=out_shape)(
        nodes, names, types, W1[:F], W1[F:], W1[:H], W2[H:H + F], W2[H + F:])


def _edge_body(pre_ref, t_ref, type1_ref, b1_ref, w2e_ref, b2_ref,
               es_ref, ep_ref):
    f32 = jnp.float32
    be = pre_ref.shape[0]
    T = type1_ref.shape[0]
    tb = t_ref[0, 0, :]
    oh = (tb[:, None] == lax.broadcasted_iota(jnp.int32, (be, T), 1)).astype(f32)
    typ = jnp.dot(oh, type1_ref[...], preferred_element_type=f32)
    pre = jnp.maximum(pre_ref[...], typ) + b1_ref[...]
    es = _leaky(pre)
    es_ref[...] = es
    ep_ref[...] = jnp.dot(es, w2e_ref[...], preferred_element_type=f32) + b2_ref[...]


def _edge_update(premax, t_idx, type1, b1, W2e, b2, BE=16000):
    E = t_idx.shape[0]
    T = type1.shape[0]
    nb = E // BE
    f32 = jnp.float32
    t3 = t_idx.reshape(nb, 1, BE)
    grid = (nb,)
    out_shape = [jax.ShapeDtypeStruct((E, H), f32),
                 jax.ShapeDtypeStruct((E, H), f32)]
    return pl.pallas_call(
        _edge_body,
        grid=grid,
        in_specs=[
            pl.BlockSpec((BE, H), lambda i: (i, 0)),
            pl.BlockSpec((1, 1, BE), lambda i: (i, 0, 0)),
            pl.BlockSpec((T, H), lambda i: (0, 0)),
            pl.BlockSpec((1, H), lambda i: (0, 0)),
            pl.BlockSpec((H, H), lambda i: (0, 0)),
            pl.BlockSpec((1, H), lambda i: (0, 0)),
        ],
        out_specs=[
            pl.BlockSpec((BE, H), lambda i: (i, 0)),
            pl.BlockSpec((BE, H), lambda i: (i, 0)),
        ],
        out_shape=out_shape,
    )(premax, t3, type1, b1.reshape(1, H), W2e, b2.reshape(1, H))


def _segmax_call(eid_p, nm_p, nd_p, qtab, name1, node1, Epad, CE, CPT):
    f32 = jnp.float32
    i32 = jnp.int32
    mesh = plsc.VectorSubcoreMesh(core_axis_name="c", subcore_axis_name="s")

    @functools.partial(
        pl.kernel,
        out_type=jax.ShapeDtypeStruct((Epad, H), f32),
        mesh=mesh,
        scratch_types=[
            pltpu.VMEM((48,), i32),
            pltpu.VMEM((C,), i32),
            pltpu.VMEM((C,), i32),
            pltpu.VMEM((C,), i32),
            pltpu.VMEM((32, H), f32),
            pltpu.VMEM((C, H), f32),
            pltpu.VMEM((CE, H), f32),
            pltpu.SemaphoreType.DMA,
            pltpu.SemaphoreType.DMA,
        ],
    )
    def seg(eid_hbm, nm_hbm, nd_hbm, qtab_hbm, name1_hbm, node1_hbm, pre_hbm,
            qbuf, eidb, nmb, ndb, name1loc, noderows, out_local, sem1, sem2):
        wid = lax.axis_index("s") * NC + lax.axis_index("c")
        pltpu.sync_copy(qtab_hbm.at[wid], qbuf)
        pltpu.sync_copy(name1_hbm, name1loc)
        neg16 = jnp.full((NLANE,), NEG, f32)

        def chunk_body(c, _):
            ce0 = (wid * CPT + c) * CE
            qa = qbuf[pl.ds(c, 16)][0]
            qe = qbuf[pl.ds(16 + c, 16)][0]

            def init_body(r, _):
                for j in range(NJ):
                    out_local[r, NLANE * j:NLANE * (j + 1)] = neg16
                return 0
            lax.fori_loop(0, CE, init_body, 0)

            nsub = (qe - qa + (C - 1)) >> 7

            def sub_body(s, carry):
                p = pl.multiple_of(qa + s * C, 8)
                i1 = pltpu.async_copy(eid_hbm.at[pl.ds(p, C)], eidb, sem1)
                i2 = pltpu.async_copy(nm_hbm.at[pl.ds(p, C)], nmb, sem1)
                i3 = pltpu.async_copy(nd_hbm.at[pl.ds(p, C)], ndb, sem1)
                i1.wait()
                i2.wait()
                i3.wait()
                d2 = pltpu.async_copy(node1_hbm.at[ndb], noderows, sem2)
                d2.wait()
                nv = jnp.minimum(C, qe - p)
                ngrp = (nv + 15) >> 4

                def grp_body(g, gcarry):
                    prev_le = gcarry[0]
                    acc = gcarry[1]
                    r0 = g * 16
                    egrp = eidb[pl.ds(r0, 16)]
                    nmg = nmb[pl.ds(r0, 16)]
                    for ii in range(16):
                        le = egrp[ii] - ce0
                        nmi = nmg[ii]
                        valid = (r0 + ii) < nv
                        d = jnp.where(le == prev_le, jnp.float32(0.0),
                                      jnp.float32(NEG))
                        dv = jnp.full((NLANE,), d, f32)
                        newacc = []
                        for j in range(NJ):
                            sl = slice(NLANE * j, NLANE * (j + 1))
                            z = name1loc[nmi, sl] + noderows[r0 + ii, sl]
                            newacc.append(jnp.maximum(acc[j] + dv, z))
                        acc = tuple(newacc)

                        @pl.when(jnp.logical_and(valid, le >= 0))
                        def _(acc=acc, le=le):
                            for j in range(NJ):
                                out_local[le, NLANE * j:NLANE * (j + 1)] = acc[j]
                        prev_le = le
                    return (prev_le, acc)

                return lax.fori_loop(0, ngrp, grp_body, carry)

            acc0 = tuple(neg16 for _ in range(NJ))
            lax.fori_loop(0, nsub, sub_body, (jnp.int32(-2147483600), acc0))
            pltpu.sync_copy(out_local, pre_hbm.at[pl.ds(ce0, CE)])
            return 0

        lax.fori_loop(0, CPT, chunk_body, 0)

    return seg(eid_p, nm_p, nd_p, qtab, name1, node1)


def _final_call(eid_p, nm_p, nd_p, ep2b, name2, node2, A):
    f32 = jnp.float32
    i32 = jnp.int32
    APT = A // NW
    NSUB = (APT + C - 1) // C
    NITER = 2 * ((NSUB + 1) // 2)
    LASTO = APT - C
    mesh = plsc.VectorSubcoreMesh(core_axis_name="c", subcore_axis_name="s")

    @functools.partial(
        pl.kernel,
        out_type=jax.ShapeDtypeStruct((A, H), f32),
        mesh=mesh,
        scratch_types=[
            [pltpu.VMEM((C,), i32)] * 2,
            [pltpu.VMEM((C,), i32)] * 2,
            [pltpu.VMEM((C,), i32)] * 2,
            [pltpu.VMEM((C, H), f32)] * 2,
            pltpu.VMEM((32, H), f32),
            [pltpu.VMEM((C, H), f32)] * 2,
            [pltpu.VMEM((C, H), f32)] * 2,
            [pltpu.SemaphoreType.DMA] * 2,
            [pltpu.SemaphoreType.DMA] * 2,
            [pltpu.SemaphoreType.DMA] * 2,
        ],
    )
    def fin(eid_hbm, nm_hbm, nd_hbm, ep_hbm, name2_hbm, node2_hbm, out_hbm,
            eidb, nmb, ndb, eprows, name2loc, noderows, outloc,
            semi, semg, semst):
        wid = lax.axis_index("s") * NC + lax.axis_index("c")
        base = wid * APT

        def off(k):
            return pl.multiple_of(base + jnp.minimum(k * C, LASTO), 8)

        def idx_copies(k, s):
            o = off(k)
            return (pltpu.make_async_copy(eid_hbm.at[pl.ds(o, C)], eidb[s], semi[s]),
                    pltpu.make_async_copy(nm_hbm.at[pl.ds(o, C)], nmb[s], semi[s]),
                    pltpu.make_async_copy(nd_hbm.at[pl.ds(o, C)], ndb[s], semi[s]))

        def gathers(s):
            return (pltpu.make_async_copy(ep_hbm.at[eidb[s]], eprows[s], semg[s]),
                    pltpu.make_async_copy(node2_hbm.at[ndb[s]], noderows[s], semg[s]))

        def store(k, s):
            return pltpu.make_async_copy(outloc[s], out_hbm.at[pl.ds(off(k), C)],
                                         semst[s])

        def issue(ds):
            for d in ds:
                d.start()

        def drain(ds):
            for d in ds:
                d.wait()

        pltpu.sync_copy(name2_hbm, name2loc)
        drain_issue0 = idx_copies(0, 0)
        issue(drain_issue0)
        drain(drain_issue0)
        issue(gathers(0))
        issue(idx_copies(1, 1))

        def pair_body(p, _):
            for s in range(2):
                k = 2 * p + s
                drain(gathers(s))

                def grp_body(g, _):
                    r0 = g * 16
                    nmg = nmb[s][pl.ds(r0, 16)]
                    for ii in range(16):
                        i = r0 + ii
                        nmi = nmg[ii]
                        for j in range(NJ):
                            sl = slice(NLANE * j, NLANE * (j + 1))
                            x = (eprows[s][i, sl] + name2loc[nmi, sl]
                                 + noderows[s][i, sl])
                            outloc[s][i, sl] = jnp.maximum(x, 0.01 * x)
                    return 0
                lax.fori_loop(0, C // 16, grp_body, 0)
                issue((store(k, s),))
                issue(idx_copies(k + 2, s))
                drain(idx_copies(k + 1, 1 - s))

                @pl.when(k >= 1)
                def _():
                    drain((store(k - 1, 1 - s),))
                issue(gathers(1 - s))
            return 0

        lax.fori_loop(0, NITER // 2, pair_body, 0)
        drain(gathers(0))
        drain(idx_copies(NITER + 1, 1))
        drain((store(NITER - 1, 1),))

    return fin(eid_p, nm_p, nd_p, ep2b, name2, node2)


def kernel(nodes_representations, hyperedge_arg_node_idxs,
           unq_hyperedge_type_reprs, hyperedge_type_name_unq_idxs,
           unq_hyperedge_arg_name_reprs, hyperedge_arg_name_unq_idxs,
           hyperedge_arg_to_edge_id, num_edges, W1, b1, W2, b2):
    A = hyperedge_arg_to_edge_id.shape[0]
    E = hyperedge_type_name_unq_idxs.shape[0]
    i32 = jnp.int32

    CE = 256
    CPT = -(-E // (NW * CE))
    NCHUNK = NW * CPT
    Epad = NCHUNK * CE

    node1, node2, name1, name2, type1 = _make_tables(
        nodes_representations, unq_hyperedge_arg_name_reprs,
        unq_hyperedge_type_reprs, W1, W2)

    eid = hyperedge_arg_to_edge_id
    bounds = jnp.arange(0, Epad + 1, CE, dtype=i32)
    q = jnp.searchsorted(eid, bounds[:NCHUNK + 1], side="left").astype(i32)
    qa = (q[:-1] // 8) * 8
    qtab = jnp.concatenate([
        jnp.pad(qa.reshape(NW, CPT), ((0, 0), (0, 16 - CPT))),
        jnp.pad(q[1:].reshape(NW, CPT), ((0, 0), (0, 32 - CPT))),
    ], axis=1)

    pad = jnp.zeros((C,), i32)
    eid_p = jnp.concatenate([eid, pad])
    nm_p = jnp.concatenate([hyperedge_arg_name_unq_idxs, pad])
    nd_p = jnp.concatenate([hyperedge_arg_node_idxs, pad])

    premax = _segmax_call(eid_p, nm_p, nd_p, qtab, name1, node1, Epad, CE, CPT)

    edge_states, ep2b = _edge_update(
        premax, hyperedge_type_name_unq_idxs, type1, b1, W2[:H], b2)

    msgs = _final_call(eid_p, nm_p, nd_p, ep2b, name2, node2, A)

    return (msgs, edge_states)


# speedup vs baseline: 3.1110x; 1.0282x over previous
import functools

import jax
import jax.numpy as jnp
from jax import lax
from jax.experimental import pallas as pl
from jax.experimental.pallas import tpu as pltpu
from jax.experimental.pallas import tpu_sc as plsc

H = 128
F = 16
NLANE = 16
NJ = H // NLANE
NEG = float(-3.0e38)

NC = 2
NS = 16
NW = NC * NS

C = 128


def _leaky(x):
    return jnp.maximum(x, 0.01 * x)


def _tables_body(nodes_ref, names_ref, types_ref, w1n_ref, w1d_ref, w1t_ref,
                 w2n_ref, w2d_ref, node1_ref, node2_ref, name1_ref, name2_ref,
                 type1_ref):
    f32 = jnp.float32
    nodes = nodes_ref[...]
    node1_ref[...] = jnp.dot(nodes, w1d_ref[...], preferred_element_type=f32)
    node2_ref[...] = jnp.dot(nodes, w2d_ref[...], preferred_element_type=f32)
    name1_ref[...] = jnp.dot(names_ref[...], w1n_ref[...], preferred_element_type=f32)
    name2_ref[...] = jnp.dot(names_ref[...], w2n_ref[...], preferred_element_type=f32)
    type1_ref[...] = jnp.dot(types_ref[...], w1t_ref[...], preferred_element_type=f32)


def _make_tables(nodes, names, types, W1, W2):
    N = nodes.shape[0]
    T = types.shape[0]
    NAMES = names.shape[0]
    f32 = jnp.float32
    out_shape = [
        jax.ShapeDtypeStruct((N, H), f32),
        jax.ShapeDtypeStruct((N, H), f32),
        jax.ShapeDtypeStruct((NAMES, H), f32),
        jax.ShapeDtypeStruct((NAMES, H), f32),
        jax.ShapeDtypeStruct((T, H), f32),
    ]
    return pl.pallas_call(_tables_body, out_shape=out_shape)(
        nodes, names, types, W1[:F], W1[F:], W1[:H], W2[H:H + F], W2[H + F:])


def _edge_body(pre_ref, t_ref, type1_ref, b1_ref, w2e_ref, b2_ref,
               es_ref, ep_ref):
    f32 = jnp.float32
    be = pre_ref.shape[0]
    T = type1_ref.shape[0]
    tb = t_ref[0, 0, :]
    oh = (tb[:, None] == lax.broadcasted_iota(jnp.int32, (be, T), 1)).astype(f32)
    typ = jnp.dot(oh, type1_ref[...], preferred_element_type=f32)
    pre = jnp.maximum(pre_ref[...], typ) + b1_ref[...]
    es = _leaky(pre)
    es_ref[...] = es
    ep_ref[...] = jnp.dot(es, w2e_ref[...], preferred_element_type=f32) + b2_ref[...]


def _edge_update(premax, t_idx, type1, b1, W2e, b2, BE=16000):
    E = t_idx.shape[0]
    T = type1.shape[0]
    nb = E // BE
    f32 = jnp.float32
    t3 = t_idx.reshape(nb, 1, BE)
    grid = (nb,)
    out_shape = [jax.ShapeDtypeStruct((E, H), f32),
                 jax.ShapeDtypeStruct((E, H), f32)]
    return pl.pallas_call(
        _edge_body,
        grid=grid,
        in_specs=[
            pl.BlockSpec((BE, H), lambda i: (i, 0)),
            pl.BlockSpec((1, 1, BE), lambda i: (i, 0, 0)),
            pl.BlockSpec((T, H), lambda i: (0, 0)),
            pl.BlockSpec((1, H), lambda i: (0, 0)),
            pl.BlockSpec((H, H), lambda i: (0, 0)),
            pl.BlockSpec((1, H), lambda i: (0, 0)),
        ],
        out_specs=[
            pl.BlockSpec((BE, H), lambda i: (i, 0)),
            pl.BlockSpec((BE, H), lambda i: (i, 0)),
        ],
        out_shape=out_shape,
    )(premax, t3, type1, b1.reshape(1, H), W2e, b2.reshape(1, H))


def _segmax_call(eid_p, nm_p, nd_p, qtab, name1, node1, Epad, CE, CPT):
    f32 = jnp.float32
    i32 = jnp.int32
    mesh = plsc.VectorSubcoreMesh(core_axis_name="c", subcore_axis_name="s")

    @functools.partial(
        pl.kernel,
        out_type=jax.ShapeDtypeStruct((Epad, H), f32),
        mesh=mesh,
        scratch_types=[
            pltpu.VMEM((48,), i32),
            [pltpu.VMEM((C,), i32)] * 2,
            [pltpu.VMEM((C,), i32)] * 2,
            [pltpu.VMEM((C,), i32)] * 2,
            pltpu.VMEM((32, H), f32),
            [pltpu.VMEM((C, H), f32)] * 2,
            pltpu.VMEM((CE, H), f32),
            pltpu.VMEM((H,), f32),
            pltpu.VMEM((NLANE,), i32),
            [pltpu.SemaphoreType.DMA] * 2,
            [pltpu.SemaphoreType.DMA] * 2,
        ],
    )
    def seg(eid_hbm, nm_hbm, nd_hbm, qtab_hbm, name1_hbm, node1_hbm, pre_hbm,
            qbuf, eidb, nmb, ndb, name1loc, noderows, out_local, accbuf,
            prevbuf, semi, semg):
        wid = lax.axis_index("s") * NC + lax.axis_index("c")
        pltpu.sync_copy(qtab_hbm.at[wid], qbuf)
        pltpu.sync_copy(name1_hbm, name1loc)
        neg16 = jnp.full((NLANE,), NEG, f32)

        def chunk_body(c, _):
            ce0 = (wid * CPT + c) * CE
            qa = qbuf[pl.ds(c, 16)][0]
            qe = qbuf[pl.ds(16 + c, 16)][0]

            def init_body(r, _):
                for j in range(NJ):
                    out_local[r, NLANE * j:NLANE * (j + 1)] = neg16
                return 0
            lax.fori_loop(0, CE, init_body, 0)

            nsub = (qe - qa + (C - 1)) >> 7

            def idx_copies(k, s):
                p = pl.multiple_of(qa + k * C, 8)
                return (pltpu.make_async_copy(eid_hbm.at[pl.ds(p, C)],
                                              eidb[s], semi[s]),
                        pltpu.make_async_copy(nm_hbm.at[pl.ds(p, C)],
                                              nmb[s], semi[s]),
                        pltpu.make_async_copy(nd_hbm.at[pl.ds(p, C)],
                                              ndb[s], semi[s]))

            def gather(s):
                return pltpu.make_async_copy(node1_hbm.at[ndb[s]],
                                             noderows[s], semg[s])

            def issue(ds):
                for d in ds:
                    d.start()

            def drain(ds):
                for d in ds:
                    d.wait()

            @pl.when(nsub > 0)
            def _():
                ds = idx_copies(0, 0)
                issue(ds)
                drain(ds)
                issue((gather(0),))

            @pl.when(nsub > 1)
            def _():
                issue(idx_copies(1, 1))

            def make_compute(s):
                def compute(k):
                    carry = (prevbuf[pl.ds(0, NLANE)][0],
                             tuple(accbuf[NLANE * j:NLANE * (j + 1)]
                                   for j in range(NJ)))
                    p = qa + k * C
                    nv = jnp.minimum(C, qe - p)
                    ngrp = (nv + 15) >> 4

                    def grp_body(g, gcarry):
                        prev_le = gcarry[0]
                        acc = gcarry[1]
                        r0 = g * 16
                        egrp = eidb[s][pl.ds(r0, 16)]
                        nmg = nmb[s][pl.ds(r0, 16)]
                        for ii in range(16):
                            le = egrp[ii] - ce0
                            nmi = nmg[ii]
                            valid = (r0 + ii) < nv
                            d = jnp.where(le == prev_le, jnp.float32(0.0),
                                          jnp.float32(NEG))
                            dv = jnp.full((NLANE,), d, f32)
                            newacc = []
                            for j in range(NJ):
                                sl = slice(NLANE * j, NLANE * (j + 1))
                                z = name1loc[nmi, sl] + noderows[s][r0 + ii, sl]
                                newacc.append(jnp.maximum(acc[j] + dv, z))
                            acc = tuple(newacc)

                            @pl.when(jnp.logical_and(valid, le >= 0))
                            def _(acc=acc, le=le):
                                for j in range(NJ):
                                    out_local[le,
                                              NLANE * j:NLANE * (j + 1)] = acc[j]
                            prev_le = le
                        return (prev_le, acc)

                    prev_le, acc = lax.fori_loop(0, ngrp, grp_body, carry)
                    prevbuf[...] = jnp.full((NLANE,), prev_le, jnp.int32)
                    for j in range(NJ):
                        accbuf[NLANE * j:NLANE * (j + 1)] = acc[j]
                return compute

            def pair_body(qq, _):
                for s in range(2):
                    k = 2 * qq + s

                    @pl.when(k < nsub)
                    def _(k=k, s=s):
                        drain((gather(s),))
                        make_compute(s)(k)

                    @pl.when(k + 2 < nsub)
                    def _(k=k, s=s):
                        issue(idx_copies(k + 2, s))

                    @pl.when(k + 1 < nsub)
                    def _(k=k, s=s):
                        drain(idx_copies(k + 1, 1 - s))
                        issue((gather(1 - s),))
                return 0

            prevbuf[...] = jnp.full((NLANE,), -2147483600, jnp.int32)
            for j in range(NJ):
                accbuf[NLANE * j:NLANE * (j + 1)] = neg16
            npair = (nsub + 1) >> 1
            lax.fori_loop(0, npair, pair_body, 0)
            pltpu.sync_copy(out_local, pre_hbm.at[pl.ds(ce0, CE)])
            return 0

        lax.fori_loop(0, CPT, chunk_body, 0)

    return seg(eid_p, nm_p, nd_p, qtab, name1, node1)


def _final_call(eid_p, nm_p, nd_p, ep2b, name2, node2, A):
    f32 = jnp.float32
    i32 = jnp.int32
    APT = A // NW
    NSUB = (APT + C - 1) // C
    NITER = 2 * ((NSUB + 1) // 2)
    LASTO = APT - C
    mesh = plsc.VectorSubcoreMesh(core_axis_name="c", subcore_axis_name="s")

    @functools.partial(
        pl.kernel,
        out_type=jax.ShapeDtypeStruct((A, H), f32),
        mesh=mesh,
        scratch_types=[
            [pltpu.VMEM((C,), i32)] * 2,
            [pltpu.VMEM((C,), i32)] * 2,
            [pltpu.VMEM((C,), i32)] * 2,
            [pltpu.VMEM((C, H), f32)] * 2,
            pltpu.VMEM((32, H), f32),
            [pltpu.VMEM((C, H), f32)] * 2,
            [pltpu.VMEM((C, H), f32)] * 2,
            [pltpu.SemaphoreType.DMA] * 2,
            [pltpu.SemaphoreType.DMA] * 2,
            [pltpu.SemaphoreType.DMA] * 2,
        ],
    )
    def fin(eid_hbm, nm_hbm, nd_hbm, ep_hbm, name2_hbm, node2_hbm, out_hbm,
            eidb, nmb, ndb, eprows, name2loc, noderows, outloc,
            semi, semg, semst):
        wid = lax.axis_index("s") * NC + lax.axis_index("c")
        base = wid * APT

        def off(k):
            return pl.multiple_of(base + jnp.minimum(k * C, LASTO), 8)

        def idx_copies(k, s):
            o = off(k)
            return (pltpu.make_async_copy(eid_hbm.at[pl.ds(o, C)], eidb[s], semi[s]),
                    pltpu.make_async_copy(nm_hbm.at[pl.ds(o, C)], nmb[s], semi[s]),
                    pltpu.make_async_copy(nd_hbm.at[pl.ds(o, C)], ndb[s], semi[s]))

        def gathers(s):
            return (pltpu.make_async_copy(ep_hbm.at[eidb[s]], eprows[s], semg[s]),
                    pltpu.make_async_copy(node2_hbm.at[ndb[s]], noderows[s], semg[s]))

        def store(k, s):
            return pltpu.make_async_copy(outloc[s], out_hbm.at[pl.ds(off(k), C)],
                                         semst[s])

        def issue(ds):
            for d in ds:
                d.start()

        def drain(ds):
            for d in ds:
                d.wait()

        pltpu.sync_copy(name2_hbm, name2loc)
        drain_issue0 = idx_copies(0, 0)
        issue(drain_issue0)
        drain(drain_issue0)
        issue(gathers(0))
        issue(idx_copies(1, 1))

        def pair_body(p, _):
            for s in range(2):
                k = 2 * p + s
                drain(gathers(s))

                def grp_body(g, _):
                    r0 = g * 16
                    nmg = nmb[s][pl.ds(r0, 16)]
                    for ii in range(16):
                        i = r0 + ii
                        nmi = nmg[ii]
                        for j in range(NJ):
                            sl = slice(NLANE * j, NLANE * (j + 1))
                            x = (eprows[s][i, sl] + name2loc[nmi, sl]
                                 + noderows[s][i, sl])
                            outloc[s][i, sl] = jnp.maximum(x, 0.01 * x)
                    return 0
                lax.fori_loop(0, C // 16, grp_body, 0)
                issue((store(k, s),))
                issue(idx_copies(k + 2, s))
                drain(idx_copies(k + 1, 1 - s))

                @pl.when(k >= 1)
                def _():
                    drain((store(k - 1, 1 - s),))
                issue(gathers(1 - s))
            return 0

        lax.fori_loop(0, NITER // 2, pair_body, 0)
        drain(gathers(0))
        drain(idx_copies(NITER + 1, 1))
        drain((store(NITER - 1, 1),))

    return fin(eid_p, nm_p, nd_p, ep2b, name2, node2)


def kernel(nodes_representations, hyperedge_arg_node_idxs,
           unq_hyperedge_type_reprs, hyperedge_type_name_unq_idxs,
           unq_hyperedge_arg_name_reprs, hyperedge_arg_name_unq_idxs,
           hyperedge_arg_to_edge_id, num_edges, W1, b1, W2, b2):
    A = hyperedge_arg_to_edge_id.shape[0]
    E = hyperedge_type_name_unq_idxs.shape[0]
    i32 = jnp.int32

    CE = 256
    CPT = -(-E // (NW * CE))
    NCHUNK = NW * CPT
    Epad = NCHUNK * CE

    node1, node2, name1, name2, type1 = _make_tables(
        nodes_representations, unq_hyperedge_arg_name_reprs,
        unq_hyperedge_type_reprs, W1, W2)

    eid = hyperedge_arg_to_edge_id
    bounds = jnp.arange(0, Epad + 1, CE, dtype=i32)
    q = jnp.searchsorted(eid, bounds[:NCHUNK + 1], side="left").astype(i32)
    qa = (q[:-1] // 8) * 8
    qtab = jnp.concatenate([
        jnp.pad(qa.reshape(NW, CPT), ((0, 0), (0, 16 - CPT))),
        jnp.pad(q[1:].reshape(NW, CPT), ((0, 0), (0, 32 - CPT))),
    ], axis=1)

    pad = jnp.zeros((C,), i32)
    eid_p = jnp.concatenate([eid, pad])
    nm_p = jnp.concatenate([hyperedge_arg_name_unq_idxs, pad])
    nd_p = jnp.concatenate([hyperedge_arg_node_idxs, pad])

    premax = _segmax_call(eid_p, nm_p, nd_p, qtab, name1, node1, Epad, CE, CPT)

    edge_states, ep2b = _edge_update(
        premax, hyperedge_type_name_unq_idxs, type1, b1, W2[:H], b2)

    msgs = _final_call(eid_p, nm_p, nd_p, ep2b, name2, node2, A)

    return (msgs, edge_states)
